# Initial kernel scaffold; baseline (speedup 1.0000x reference)
import jax
import jax.numpy as jnp
from jax.experimental import pallas as pl


def kernel(counts_slvra, weights_slvra, source_weights_s, flat_idx, sources):
    raise NotImplementedError("write your pallas kernel here")


# speedup vs baseline: 30.7371x; 30.7371x over previous
import functools

import jax
import jax.numpy as jnp
from jax import lax
from jax.experimental import pallas as pl
from jax.experimental.pallas import tpu as pltpu
from jax.experimental.pallas import tpu_sc as plsc

S, L, V, R, A = 4, 3, 6, 10, 10
TABLE = S * L * V * R * A
N = 2_000_000
ATT = 0.99999 ** N

NC, NS, LANES = 2, 16, 16
NW = NC * NS
PER_W = 62_528
NPAD = NW * PER_W
SENTINEL = 7200
ROWS = 80
TBL = ROWS * 128
CH = PER_W // 4

_mesh = functools.partial(
    plsc.VectorSubcoreMesh, core_axis_name="c", subcore_axis_name="s"
)


_sc_params = pltpu.CompilerParams(needs_layout_passes=False)


@functools.partial(
    pl.kernel,
    mesh=_mesh(),
    out_type=jax.ShapeDtypeStruct((NW, TBL), jnp.float32),
    scratch_types=[
        pltpu.VMEM((PER_W,), jnp.int32),
        pltpu.VMEM((TBL,), jnp.float32),
    ],
    compiler_params=_sc_params,
)
def _hist_kernel(idx_hbm, out_hbm, idx_v, tbl_v):
    wid = lax.axis_index("s") * NC + lax.axis_index("c")
    base = wid * PER_W
    pltpu.sync_copy(idx_hbm.at[pl.ds(base, PER_W)], idx_v)

    def zero_body(k, c):
        tbl_v[pl.ds(k * LANES, LANES)] = jnp.zeros((LANES,), jnp.float32)
        return c

    lax.fori_loop(0, TBL // LANES, zero_body, 0)

    ones = jnp.ones((LANES,), jnp.float32)

    def body(i, c):
        iv = idx_v[pl.ds(i * LANES, LANES)]
        j = iv + 28 * (iv // 100)
        plsc.addupdate_scatter(tbl_v, [j], ones)
        return c

    lax.fori_loop(0, PER_W // LANES, body, 0)
    pltpu.sync_copy(tbl_v, out_hbm.at[wid])


def _table_kernel(part_ref, c0_ref, w0_ref, sw0_ref, wout_ref, swout_ref):
    acc = c0_ref[...]
    for i in range(NW):
        acc = acc + part_ref[i]
    rows = []
    cs = []
    for s in range(S):
        art = acc[s * 18 : s * 18 + 6]
        var = acc[s * 18 + 6 : s * 18 + 12]
        unl = acc[s * 18 + 12 : s * 18 + 18]
        ratio = (art + 0.01) / (var + 0.01)
        w_art = jnp.clip((1.0 + 1.0 / ratio) * 0.5, 0.01, 100.0)
        w_var = jnp.clip((1.0 + ratio) * 0.5, 0.01, 100.0)
        sa = jnp.sum(art, axis=1, keepdims=True)
        su = jnp.sum(unl, axis=1, keepdims=True)
        w_unl = jnp.broadcast_to(jnp.clip((sa + sa) / su, 0.0, 1.0), (6, 128))
        rows += [w_art, w_var, w_unl]
        cs.append(jnp.sum(acc[s * 18 : (s + 1) * 18]))
    neww = jnp.concatenate(rows + [jnp.zeros((8, 128), jnp.float32)], axis=0)
    wout_ref[...] = ATT * w0_ref[...] + (1.0 - ATT) * neww

    total = cs[0] + cs[1] + cs[2] + cs[3]
    row_i = lax.broadcasted_iota(jnp.int32, (8, 128), 0)
    col_i = lax.broadcasted_iota(jnp.int32, (8, 128), 1)
    swv = jnp.zeros((8, 128), jnp.float32)
    for s in range(S):
        sw_s = ATT * sw0_ref[0, s] + (1.0 - ATT) * (total / cs[s] / S)
        swv = jnp.where((row_i == 0) & (col_i == s), sw_s, swv)
    swout_ref[...] = swv


@functools.partial(
    pl.kernel,
    mesh=_mesh(),
    out_type=(
        jax.ShapeDtypeStruct((NPAD,), jnp.float32),
        jax.ShapeDtypeStruct((NPAD,), jnp.float32),
    ),
    scratch_types=[
        pltpu.VMEM((CH,), jnp.int32),
        pltpu.VMEM((CH,), jnp.float32),
        pltpu.VMEM((CH,), jnp.float32),
        pltpu.VMEM((TBL,), jnp.float32),
        pltpu.VMEM((LANES,), jnp.float32),
    ],
    compiler_params=_sc_params,
)
def _gather_kernel(idx_hbm, wtab_hbm, swtab_hbm, bw_hbm, swb_hbm,
                   idx_v, bw_v, sw_v, tbl_v, swt_v):
    wid = lax.axis_index("s") * NC + lax.axis_index("c")
    pltpu.sync_copy(wtab_hbm, tbl_v)
    pltpu.sync_copy(swtab_hbm, swt_v)
    for ch in range(PER_W // CH):
        base = wid * PER_W + ch * CH

        pltpu.sync_copy(idx_hbm.at[pl.ds(base, CH)], idx_v)

        def body(i, c):
            iv = idx_v[pl.ds(i * LANES, LANES)]
            j = iv + 28 * (iv // 100)
            bw_v[pl.ds(i * LANES, LANES)] = plsc.load_gather(tbl_v, [j])
            sw_v[pl.ds(i * LANES, LANES)] = plsc.load_gather(swt_v, [iv // 1800])
            return c

        lax.fori_loop(0, CH // LANES, body, 0)
        pltpu.sync_copy(bw_v, bw_hbm.at[pl.ds(base, CH)])
        pltpu.sync_copy(sw_v, swb_hbm.at[pl.ds(base, CH)])


def kernel(counts_slvra, weights_slvra, source_weights_s, flat_idx, sources):
    del sources
    idx_p = jnp.concatenate(
        [flat_idx.astype(jnp.int32),
         jnp.full((NPAD - N,), SENTINEL, jnp.int32)])

    partials = _hist_kernel(idx_p)

    c0 = jnp.pad(counts_slvra.reshape(72, 100), ((0, 8), (0, 28)))
    w0 = jnp.pad(weights_slvra.reshape(72, 100), ((0, 8), (0, 28)))
    sw0 = jnp.zeros((8, 128), jnp.float32).at[0, :4].set(source_weights_s)

    wtab, swout = pl.pallas_call(
        _table_kernel,
        out_shape=[
            jax.ShapeDtypeStruct((ROWS, 128), jnp.float32),
            jax.ShapeDtypeStruct((8, 128), jnp.float32),
        ],
    )(partials.reshape(NW, ROWS, 128), c0, w0, sw0)

    bw_p, swb_p = _gather_kernel(idx_p, wtab.reshape(TBL), swout[0, :LANES])
    return bw_p[:N], swb_p[:N]
